# baseline (speedup 1.0000x reference)
import functools

import jax
import jax.numpy as jnp
from jax import lax
from jax.experimental import pallas as pl
from jax.experimental.pallas import tpu as pltpu
from jax.experimental.pallas import tpu_sc as plsc

BATCH = 16384
EMBED = 64
NUM_TABLES = 3
NW = 32
BPW = BATCH // NW

_mesh = plsc.VectorSubcoreMesh(core_axis_name="c", subcore_axis_name="s")


@functools.partial(
    pl.kernel,
    mesh=_mesh,
    compiler_params=pltpu.CompilerParams(use_tc_tiling_on_sc=False),
    out_type=jax.ShapeDtypeStruct((BATCH, NUM_TABLES * EMBED), jnp.float32),
    scratch_types=[
        pltpu.VMEM((BPW,), jnp.int32),
        pltpu.VMEM((BPW,), jnp.int32),
        pltpu.VMEM((BPW,), jnp.int32),
        pltpu.VMEM((BPW, EMBED), jnp.float32),
        pltpu.VMEM((BPW, EMBED), jnp.float32),
        pltpu.VMEM((BPW, EMBED), jnp.float32),
        pltpu.SemaphoreType.DMA,
        pltpu.SemaphoreType.DMA,
        pltpu.SemaphoreType.DMA,
    ],
)
def _emb_kernel(xT_hbm, mi_hbm, mo_hbm, mtext_hbm, out_hbm,
                idx0, idx1, idx2, r0, r1, r2, s0, s1, s2):
    wid = lax.axis_index("s") * 2 + lax.axis_index("c")
    base = wid * BPW
    pltpu.sync_copy(xT_hbm.at[pl.ds(base, BPW)], idx0)
    pltpu.sync_copy(xT_hbm.at[pl.ds(BATCH + base, BPW)], idx1)
    pltpu.sync_copy(xT_hbm.at[pl.ds(2 * BATCH + base, BPW)], idx2)
    c0 = pltpu.async_copy(mi_hbm.at[idx0], r0, s0)
    c1 = pltpu.async_copy(mo_hbm.at[idx1], r1, s1)
    c2 = pltpu.async_copy(mtext_hbm.at[idx2], r2, s2)
    c0.wait()
    pltpu.sync_copy(r0, out_hbm.at[pl.ds(base, BPW), pl.ds(0, EMBED)])
    c1.wait()
    pltpu.sync_copy(r1, out_hbm.at[pl.ds(base, BPW), pl.ds(EMBED, EMBED)])
    c2.wait()
    pltpu.sync_copy(r2, out_hbm.at[pl.ds(base, BPW), pl.ds(2 * EMBED, EMBED)])


def kernel(x, emb_mi, emb_mo, emb_mtext):
    xT = x.T.reshape(NUM_TABLES * BATCH)
    return _emb_kernel(xT, emb_mi, emb_mo, emb_mtext)
